# speedup vs baseline: 6.4216x; 6.4216x over previous
import functools

import jax
import jax.numpy as jnp
from jax import lax
from jax.experimental import pallas as pl
from jax.experimental.pallas import tpu as pltpu
from jax.experimental.pallas import tpu_sc as plsc

_NW = 32
_NC = 2
_L = 128
_R = 2


def kernel(x, table):
    B, H = x.shape
    V, D = table.shape
    N = B * H
    n_rows = N // _L
    rows_pw = n_rows // _NW
    chunks = rows_pw // _R
    G = _R * _L

    idx2d = x.reshape(n_rows, _L).astype(jnp.int32)

    mesh = plsc.VectorSubcoreMesh(core_axis_name="c", subcore_axis_name="s")

    @functools.partial(
        pl.kernel,
        mesh=mesh,
        out_type=jax.ShapeDtypeStruct((N, D), jnp.float32),
        scratch_types=[
            pltpu.VMEM((_R, _L), jnp.int32),
            pltpu.VMEM((G, D), jnp.float32),
            pltpu.SemaphoreType.DMA,
        ],
    )
    def sc_gather(idx_hbm, tab_hbm, out_hbm, idx_v, rows_v, sem):
        wid = lax.axis_index("s") * _NC + lax.axis_index("c")
        r0 = wid * rows_pw

        def body(i, carry):
            r = r0 + i * _R
            pltpu.sync_copy(idx_hbm.at[pl.ds(r, _R)], idx_v)
            handles = [
                pltpu.async_copy(
                    tab_hbm.at[idx_v.at[j]],
                    rows_v.at[pl.ds(j * _L, _L)],
                    sem,
                )
                for j in range(_R)
            ]
            for h in handles:
                h.wait()
            pltpu.sync_copy(rows_v, out_hbm.at[pl.ds(r * _L, G)])
            return carry

        lax.fori_loop(0, chunks, body, 0)

    out = sc_gather(idx2d, table)
    return out.reshape(B, H, D)


# speedup vs baseline: 6.6318x; 1.0327x over previous
import functools

import jax
import jax.numpy as jnp
from jax import lax
from jax.experimental import pallas as pl
from jax.experimental.pallas import tpu as pltpu
from jax.experimental.pallas import tpu_sc as plsc

_NW = 32
_NC = 2
_L = 128
_R = 2
_NB = 2


def kernel(x, table):
    B, H = x.shape
    V, D = table.shape
    N = B * H
    n_rows = N // _L
    rows_pw = n_rows // _NW
    chunks = rows_pw // _R
    G = _R * _L

    idx2d = x.reshape(n_rows, _L).astype(jnp.int32)

    mesh = plsc.VectorSubcoreMesh(core_axis_name="c", subcore_axis_name="s")

    @functools.partial(
        pl.kernel,
        mesh=mesh,
        out_type=jax.ShapeDtypeStruct((N, D), jnp.float32),
        scratch_types=[
            pltpu.VMEM((_NB, _R, _L), jnp.int32),
            pltpu.VMEM((_NB, G, D), jnp.float32),
            pltpu.SemaphoreType.DMA,
            pltpu.SemaphoreType.DMA,
            pltpu.SemaphoreType.DMA,
            pltpu.SemaphoreType.DMA,
            pltpu.SemaphoreType.DMA,
        ],
    )
    def sc_gather(idx_hbm, tab_hbm, out_hbm, idx_v, rows_v,
                  sem_i0, sem_i1, sem_g, sem_s0, sem_s1):
        wid = lax.axis_index("s") * _NC + lax.axis_index("c")
        r0 = wid * rows_pw
        sem_i = [sem_i0, sem_i1]
        sem_s = [sem_s0, sem_s1]

        def idx_copy(i, slot):
            return pltpu.make_async_copy(
                idx_hbm.at[pl.ds(r0 + i * _R, _R)], idx_v.at[slot],
                sem_i[slot])

        def store_copy(i, slot):
            return pltpu.make_async_copy(
                rows_v.at[slot], out_hbm.at[pl.ds((r0 + i * _R) * _L, G)],
                sem_s[slot])

        for b in range(_NB):
            idx_copy(b, b).start()

        def body(i2, carry):
            for b in range(_NB):
                i = i2 * _NB + b
                idx_copy(i, b).wait()
                @pl.when(i2 > 0)
                def _():
                    store_copy(i - _NB, b).wait()
                handles = [
                    pltpu.async_copy(
                        tab_hbm.at[idx_v.at[b, j]],
                        rows_v.at[b, pl.ds(j * _L, _L)],
                        sem_g,
                    )
                    for j in range(_R)
                ]
                for h in handles:
                    h.wait()
                @pl.when(i2 < chunks // _NB - 1)
                def _():
                    idx_copy(i + _NB, b).start()
                store_copy(i, b).start()
            return carry

        lax.fori_loop(0, chunks // _NB, body, 0)

        for b in range(_NB):
            store_copy(chunks - _NB + b, b).wait()

    out = sc_gather(idx2d, table)
    return out.reshape(B, H, D)


# speedup vs baseline: 19.0314x; 2.8697x over previous
import functools

import jax
import jax.numpy as jnp
from jax import lax
from jax.experimental import pallas as pl
from jax.experimental.pallas import tpu as pltpu
from jax.experimental.pallas import tpu_sc as plsc

_NW = 32
_NC = 2
_L = 128
_R = 2
_NB = 2


def kernel(x, table):
    B, H = x.shape
    V, D = table.shape
    N = B * H
    n_rows = N // _L
    rows_pw = n_rows // _NW
    chunks = rows_pw // _R
    G = _R * _L

    idx2d = x.reshape(n_rows, _L).astype(jnp.int32)

    mesh = plsc.VectorSubcoreMesh(core_axis_name="c", subcore_axis_name="s")

    @functools.partial(
        pl.kernel,
        mesh=mesh,
        out_type=jax.ShapeDtypeStruct((N, D), jnp.float32),
        scratch_types=[
            pltpu.VMEM((_NB, _R, _L), jnp.int32),
            pltpu.VMEM((_NB, G, D), jnp.float32),
            pltpu.VMEM_SHARED((V, D), jnp.float32),
            pltpu.SemaphoreType.DMA,
            pltpu.SemaphoreType.DMA,
            pltpu.SemaphoreType.DMA,
            pltpu.SemaphoreType.DMA,
            pltpu.SemaphoreType.DMA,
        ],
    )
    def sc_gather(idx_hbm, tab_hbm, out_hbm, idx_v, rows_v, tab_sh,
                  sem_i0, sem_i1, sem_g, sem_s0, sem_s1):
        wid = lax.axis_index("s") * _NC + lax.axis_index("c")
        r0 = wid * rows_pw
        sem_i = [sem_i0, sem_i1]
        sem_s = [sem_s0, sem_s1]

        @pl.when(lax.axis_index("s") == 0)
        def _():
            pltpu.sync_copy(tab_hbm, tab_sh)
        plsc.subcore_barrier()

        def idx_copy(i, slot):
            return pltpu.make_async_copy(
                idx_hbm.at[pl.ds(r0 + i * _R, _R)], idx_v.at[slot],
                sem_i[slot])

        def store_copy(i, slot):
            return pltpu.make_async_copy(
                rows_v.at[slot], out_hbm.at[pl.ds((r0 + i * _R) * _L, G)],
                sem_s[slot])

        for b in range(_NB):
            idx_copy(b, b).start()

        def body(i2, carry):
            for b in range(_NB):
                i = i2 * _NB + b
                idx_copy(i, b).wait()
                @pl.when(i2 > 0)
                def _():
                    store_copy(i - _NB, b).wait()
                handles = [
                    pltpu.async_copy(
                        tab_sh.at[idx_v.at[b, j]],
                        rows_v.at[b, pl.ds(j * _L, _L)],
                        sem_g,
                    )
                    for j in range(_R)
                ]
                for h in handles:
                    h.wait()
                @pl.when(i2 < chunks // _NB - 1)
                def _():
                    idx_copy(i + _NB, b).start()
                store_copy(i, b).start()
            return carry

        lax.fori_loop(0, chunks // _NB, body, 0)

        for b in range(_NB):
            store_copy(chunks - _NB + b, b).wait()

    out = sc_gather(idx2d, table)
    return out.reshape(B, H, D)


# speedup vs baseline: 19.4550x; 1.0223x over previous
import functools

import jax
import jax.numpy as jnp
from jax import lax
from jax.experimental import pallas as pl
from jax.experimental.pallas import tpu as pltpu
from jax.experimental.pallas import tpu_sc as plsc

_NW = 32
_NC = 2
_L = 128
_R = 2
_NB = 2


def kernel(x, table):
    B, H = x.shape
    V, D = table.shape
    N = B * H
    n_rows = N // _L
    rows_pw = n_rows // _NW
    chunks = rows_pw // _R
    G = _R * _L

    idx2d = x.reshape(n_rows, _L).astype(jnp.int32)

    mesh = plsc.VectorSubcoreMesh(core_axis_name="c", subcore_axis_name="s")

    @functools.partial(
        pl.kernel,
        mesh=mesh,
        out_type=jax.ShapeDtypeStruct((N, D), jnp.float32),
        scratch_types=[
            pltpu.VMEM((_NB, _R, _L), jnp.int32),
            pltpu.VMEM((_NB, G, D), jnp.float32),
            pltpu.VMEM_SHARED((V, D), jnp.float32),
            pltpu.SemaphoreType.DMA,
            pltpu.SemaphoreType.DMA,
            pltpu.SemaphoreType.DMA,
            pltpu.SemaphoreType.DMA,
            pltpu.SemaphoreType.DMA,
            pltpu.SemaphoreType.DMA,
        ],
    )
    def sc_gather(idx_hbm, tab_hbm, out_hbm, idx_v, rows_v, tab_sh,
                  sem_i0, sem_i1, sem_g0, sem_g1, sem_s0, sem_s1):
        wid = lax.axis_index("s") * _NC + lax.axis_index("c")
        r0 = wid * rows_pw
        sem_i = [sem_i0, sem_i1]
        sem_g = [sem_g0, sem_g1]
        sem_s = [sem_s0, sem_s1]

        @pl.when(lax.axis_index("s") == 0)
        def _():
            pltpu.sync_copy(tab_hbm, tab_sh)
        plsc.subcore_barrier()

        def idx_copy(i, slot):
            return pltpu.make_async_copy(
                idx_hbm.at[pl.ds(r0 + i * _R, _R)], idx_v.at[slot],
                sem_i[slot])

        def gather_copies(i, slot):
            del i
            return [
                pltpu.make_async_copy(
                    tab_sh.at[idx_v.at[slot, j]],
                    rows_v.at[slot, pl.ds(j * _L, _L)],
                    sem_g[slot])
                for j in range(_R)
            ]

        def store_copy(i, slot):
            return pltpu.make_async_copy(
                rows_v.at[slot], out_hbm.at[pl.ds((r0 + i * _R) * _L, G)],
                sem_s[slot])

        for b in range(_NB):
            idx_copy(b, b).start()
        idx_copy(0, 0).wait()
        for c in gather_copies(0, 0):
            c.start()

        def body(i2, carry):
            for b in range(_NB):
                i = i2 * _NB + b
                s = b
                s1 = 1 - b
                @pl.when(i2 * _NB + b < chunks - 1)
                def _():
                    idx_copy(i + 1, s1).wait()

                    @pl.when(i2 * _NB + b >= 1)
                    def _():
                        store_copy(i - 1, s1).wait()

                    for c in gather_copies(i + 1, s1):
                        c.start()
                for c in gather_copies(i, s):
                    c.wait()

                @pl.when(i + _NB < chunks)
                def _():
                    idx_copy(i + _NB, s).start()
                store_copy(i, s).start()
            return carry

        lax.fori_loop(0, chunks // _NB, body, 0)

        for b in range(_NB):
            store_copy(chunks - _NB + b, b).wait()

    out = sc_gather(idx2d, table)
    return out.reshape(B, H, D)
